# baseline (speedup 1.0000x reference)
import functools

import jax
import jax.numpy as jnp
from jax import lax
from jax.experimental import pallas as pl
from jax.experimental.pallas import tpu as pltpu
from jax.experimental.pallas import tpu_sc as plsc

N = 100000
E = 1600000
D = 32
DE = 16
G = 64
STEPS = 2

NC = 2
NS = 16
NWK = NC * NS
L = 16
NPW = N // NWK
W = 1600
NWIN = E // W
C = 64

NBLK = 25
BN = N // NBLK
EBLK = 200
BE = E // EBLK

_NEG_INF = float("-inf")


def _sc_agg_body(xw_hbm, eaw_hbm, src_hbm, dst_hbm, agg_hbm,
                 dstbuf, srcbuf, keptid, keptsrc, keptnl,
                 idxc, srcc, nlc, eawb, xwb, aggb,
                 sem_d, sem_s, sem_e, sem_x, sem_e2, sem_x2):
    cid = lax.axis_index("c")
    sid = lax.axis_index("s")
    wid = sid * NC + cid
    lo = wid * NPW
    io = lax.iota(jnp.int32, L)
    minf = jnp.full((L,), _NEG_INF, jnp.float32)
    zid = jnp.zeros((L,), jnp.int32)

    def _init_agg(r, _):
        aggb[r, pl.ds(0, L)] = minf
        aggb[r, pl.ds(L, L)] = minf
        return 0
    lax.fori_loop(0, NPW + 1, _init_agg, 0)

    def _init_ids(i, _):
        keptid[pl.ds(i * L, L)] = zid
        keptsrc[pl.ds(i * L, L)] = zid
        return 0
    lax.fori_loop(0, (W + L) // L, _init_ids, 0)

    def _issue(win):
        par = (win % 2) * W
        pltpu.async_copy(dst_hbm.at[pl.ds(win * W, W)],
                         dstbuf.at[pl.ds(par, W)], sem_d)
        pltpu.async_copy(src_hbm.at[pl.ds(win * W, W)],
                         srcbuf.at[pl.ds(par, W)], sem_s)

    def _wait(win):
        par = (win % 2) * W
        pltpu.make_async_copy(dst_hbm.at[pl.ds(win * W, W)],
                              dstbuf.at[pl.ds(par, W)], sem_d).wait()
        pltpu.make_async_copy(src_hbm.at[pl.ds(win * W, W)],
                              srcbuf.at[pl.ds(par, W)], sem_s).wait()

    def _gissue(slot, se, sx):
        pltpu.async_copy(eaw_hbm.at[idxc.at[slot]], eawb.at[slot], se)
        pltpu.async_copy(xw_hbm.at[srcc.at[slot]], xwb.at[slot], sx)

    def _gwait(slot, se, sx):
        pltpu.make_async_copy(eaw_hbm.at[idxc.at[slot]],
                              eawb.at[slot], se).wait()
        pltpu.make_async_copy(xw_hbm.at[srcc.at[slot]],
                              xwb.at[slot], sx).wait()

    def _consume(slot, nvalid):
        def _msg(r, _):
            for h in (0, L):
                ev = eawb[slot, r, pl.ds(h, L)]
                xv = xwb[slot, r, pl.ds(h, L)]
                sm = xv + ev
                eawb[slot, r, pl.ds(h, L)] = jnp.maximum(sm, 0.01 * sm)
            return 0
        plsc.parallel_loop(0, C, 1, unroll=4, carry=jnp.int32(0))(_msg)

        def _rmw_grp(g, _):
            nlr = nlc[slot, pl.ds(g * L, L)]
            valid = (g * L + io) < nvalid
            nlv = jnp.where(valid, nlr, NPW)
            for lane in range(L):
                nl = nlv[lane]
                row = g * L + lane
                for h in (0, L):
                    msg = eawb[slot, row, pl.ds(h, L)]
                    av = aggb[nl, pl.ds(h, L)]
                    aggb[nl, pl.ds(h, L)] = jnp.maximum(av, msg)
            return 0
        lax.fori_loop(0, C // L, _rmw_grp, 0)

    def _stage(slot, coff):
        for q in range(C // L):
            idxc[slot, pl.ds(q * L, L)] = keptid[pl.ds(coff + q * L, L)]
            srcc[slot, pl.ds(q * L, L)] = keptsrc[pl.ds(coff + q * L, L)]
            nlc[slot, pl.ds(q * L, L)] = keptnl[pl.ds(coff + q * L, L)]

    _issue(0)

    def _win_body(win, pcnt):
        wbase = win * W
        par = (win % 2) * W
        slot = win % 2
        pslot = 1 - slot
        _wait(win)

        @pl.when(win + 1 < NWIN)
        def _():
            _issue(win + 1)

        def _scan(v, cnt):
            dv = dstbuf[pl.ds(par + v * L, L)]
            sv = srcbuf[pl.ds(par + v * L, L)]
            m = (dv >= lo) & (dv < lo + NPW)
            plsc.store_compressed(keptid.at[pl.ds(cnt, L)],
                                  wbase + v * L + io, mask=m)
            plsc.store_compressed(keptsrc.at[pl.ds(cnt, L)], sv, mask=m)
            plsc.store_compressed(keptnl.at[pl.ds(cnt, L)], dv - lo, mask=m)
            pc = plsc.all_reduce_population_count(m)
            return cnt + pc[0]

        cnt = plsc.parallel_loop(0, W // L, 1, unroll=4,
                                 carry=jnp.int32(0))(_scan)

        _stage(slot, 0)

        @pl.when(pcnt > 0)
        def _():
            _gwait(pslot, sem_e, sem_x)
            _consume(pslot, jnp.minimum(pcnt, C))

        @pl.when(cnt > 0)
        def _():
            _gissue(slot, sem_e, sem_x)

        @pl.when(cnt > C)
        def _():
            nch = lax.div(cnt + (C - 1), C)

            def _chunk(c, _):
                coff = c * C
                _stage(pslot, coff)
                _gissue(pslot, sem_e2, sem_x2)
                _gwait(pslot, sem_e2, sem_x2)
                _consume(pslot, cnt - coff)
                return 0

            lax.fori_loop(1, nch, _chunk, 0)

        return cnt

    pcnt = lax.fori_loop(0, NWIN, _win_body, jnp.int32(0))

    @pl.when(pcnt > 0)
    def _():
        _gwait((NWIN - 1) % 2, sem_e, sem_x)
        _consume((NWIN - 1) % 2, jnp.minimum(pcnt, C))

    pltpu.sync_copy(aggb.at[pl.ds(0, NPW)], agg_hbm.at[wid])


def _sc_agg(xw, eaw, src, dst):
    mesh = plsc.VectorSubcoreMesh(
        core_axis_name="c", subcore_axis_name="s", num_cores=NC, num_subcores=NS)
    f = pl.kernel(
        _sc_agg_body,
        out_type=jax.ShapeDtypeStruct((NWK, NPW, D), jnp.float32),
        mesh=mesh,
        compiler_params=pltpu.CompilerParams(
            needs_layout_passes=False, use_tc_tiling_on_sc=False),
        scratch_types=[
            pltpu.VMEM((2 * W,), jnp.int32),
            pltpu.VMEM((2 * W,), jnp.int32),
            pltpu.VMEM((W + L,), jnp.int32),
            pltpu.VMEM((W + L,), jnp.int32),
            pltpu.VMEM((W + L,), jnp.int32),
            pltpu.VMEM((2, C), jnp.int32),
            pltpu.VMEM((2, C), jnp.int32),
            pltpu.VMEM((2, C), jnp.int32),
            pltpu.VMEM((2, C, D), jnp.float32),
            pltpu.VMEM((2, C, D), jnp.float32),
            pltpu.VMEM((NPW + 1, D), jnp.float32),
            pltpu.SemaphoreType.DMA,
            pltpu.SemaphoreType.DMA,
            pltpu.SemaphoreType.DMA,
            pltpu.SemaphoreType.DMA,
            pltpu.SemaphoreType.DMA,
            pltpu.SemaphoreType.DMA,
        ],
    )
    return f(xw, eaw, src, dst)


def _leaky(v):
    return jnp.maximum(v, 0.01 * v)


def _eaw_body(ea_ref, w0_ref, w1_ref, o0_ref, o1_ref):
    ea = ea_ref[...]
    o0_ref[...] = jnp.dot(ea, w0_ref[...], preferred_element_type=jnp.float32)
    o1_ref[...] = jnp.dot(ea, w1_ref[...], preferred_element_type=jnp.float32)


def _eaw(edge_attr, we0, we1):
    return pl.pallas_call(
        _eaw_body,
        grid=(EBLK,),
        in_specs=[
            pl.BlockSpec((BE, DE), lambda i: (i, 0)),
            pl.BlockSpec((DE, D), lambda i: (0, 0)),
            pl.BlockSpec((DE, D), lambda i: (0, 0)),
        ],
        out_specs=[
            pl.BlockSpec((BE, D), lambda i: (i, 0)),
            pl.BlockSpec((BE, D), lambda i: (i, 0)),
        ],
        out_shape=[
            jax.ShapeDtypeStruct((E, D), jnp.float32),
            jax.ShapeDtypeStruct((E, D), jnp.float32),
        ],
    )(edge_attr, we0, we1)


def _xw_body(x_ref, w_ref, b_ref, o_ref):
    o_ref[...] = (
        jnp.dot(x_ref[...], w_ref[...], preferred_element_type=jnp.float32)
        + b_ref[...])


def _xw(x, wx, b):
    return pl.pallas_call(
        _xw_body,
        grid=(NBLK,),
        in_specs=[
            pl.BlockSpec((BN, D), lambda i: (i, 0)),
            pl.BlockSpec((D, D), lambda i: (0, 0)),
            pl.BlockSpec((1, D), lambda i: (0, 0)),
        ],
        out_specs=pl.BlockSpec((BN, D), lambda i: (i, 0)),
        out_shape=jax.ShapeDtypeStruct((N, D), jnp.float32),
    )(x, wx, b)


def _update_body(x_ref, agg_ref, b_ref, xg_ref, wax_ref, wag_ref, waa_ref,
                 ba_ref, wg_ref, bg_ref, wxn_ref, bxn_ref,
                 xn_ref, gmax_ref, *maybe_xwn, with_next):
    xwn_ref = maybe_xwn[0] if with_next else None
    i = pl.program_id(0)
    x = x_ref[...]
    agg = agg_ref[...]
    agg = jnp.where(jnp.isfinite(agg), agg, 0.0)
    b = b_ref[0, 0]
    oh = (b[:, None] == lax.broadcasted_iota(jnp.int32, (BN, G), 1)
          ).astype(jnp.float32)
    xgb = jnp.dot(oh, xg_ref[...], preferred_element_type=jnp.float32)
    z2 = (jnp.dot(x, wax_ref[...], preferred_element_type=jnp.float32)
          + jnp.dot(xgb, wag_ref[...], preferred_element_type=jnp.float32)
          + jnp.dot(agg, waa_ref[...], preferred_element_type=jnp.float32)
          + ba_ref[...])
    xn = _leaky(z2) + x
    xn_ref[...] = xn
    gate = jnp.sum(xn * wg_ref[...], axis=1) + bg_ref[0, 0]
    gm_blk = jnp.max(jnp.where(oh > 0, gate[:, None], _NEG_INF), axis=0)

    @pl.when(i == 0)
    def _():
        gmax_ref[...] = jnp.full((8, G), _NEG_INF, jnp.float32)

    gmax_ref[...] = jnp.maximum(gmax_ref[...], jnp.broadcast_to(gm_blk, (8, G)))
    if with_next:
        xwn_ref[...] = (
            jnp.dot(xn, wxn_ref[...], preferred_element_type=jnp.float32)
            + bxn_ref[...])


def _update(x, agg, batch3, xg, wax, wag, waa, ba, wg, bg, wxn, bxn,
            with_next):
    outs = [
        jax.ShapeDtypeStruct((N, D), jnp.float32),
        jax.ShapeDtypeStruct((8, G), jnp.float32),
    ]
    out_specs = [
        pl.BlockSpec((BN, D), lambda i: (i, 0)),
        pl.BlockSpec((8, G), lambda i: (0, 0)),
    ]
    if with_next:
        outs.append(jax.ShapeDtypeStruct((N, D), jnp.float32))
        out_specs.append(pl.BlockSpec((BN, D), lambda i: (i, 0)))
    res = pl.pallas_call(
        functools.partial(_update_body, with_next=with_next),
        grid=(NBLK,),
        in_specs=[
            pl.BlockSpec((BN, D), lambda i: (i, 0)),
            pl.BlockSpec((BN, D), lambda i: (i, 0)),
            pl.BlockSpec((1, 1, BN), lambda i: (i, 0, 0)),
            pl.BlockSpec((G, D), lambda i: (0, 0)),
            pl.BlockSpec((D, D), lambda i: (0, 0)),
            pl.BlockSpec((D, D), lambda i: (0, 0)),
            pl.BlockSpec((D, D), lambda i: (0, 0)),
            pl.BlockSpec((1, D), lambda i: (0, 0)),
            pl.BlockSpec((1, D), lambda i: (0, 0)),
            pl.BlockSpec((1, 1), lambda i: (0, 0)),
            pl.BlockSpec((D, D), lambda i: (0, 0)),
            pl.BlockSpec((1, D), lambda i: (0, 0)),
        ],
        out_specs=out_specs,
        out_shape=outs,
    )(x, agg, batch3, xg, wax, wag, waa, ba, wg, bg, wxn, bxn)
    return res


def _pool_body(xn_ref, b_ref, gmax_ref, xg_ref, wf_ref, bf_ref, wg_ref,
               bg_ref, wta_ref, wtb_ref, bt_ref, xgo_ref, num_ref, den_ref):
    i = pl.program_id(0)
    xn = xn_ref[...]
    b = b_ref[0, 0]
    oh = (b[:, None] == lax.broadcasted_iota(jnp.int32, (BN, G), 1)
          ).astype(jnp.float32)
    gate = jnp.sum(xn * wg_ref[...], axis=1) + bg_ref[0, 0]
    gm = gmax_ref[...][0]
    gm = jnp.where(gm == _NEG_INF, 0.0, gm)
    gnode = jnp.dot(oh, gm[:, None], preferred_element_type=jnp.float32)[:, 0]
    e = jnp.exp(gate - gnode)
    feat = _leaky(
        jnp.dot(xn, wf_ref[...], preferred_element_type=jnp.float32)
        + bf_ref[...])
    ef = e[:, None] * feat
    num_part = lax.dot_general(oh, ef, (((0,), (0,)), ((), ())),
                               preferred_element_type=jnp.float32)
    den_part = lax.dot_general(oh, e[:, None], (((0,), (0,)), ((), ())),
                               preferred_element_type=jnp.float32)

    @pl.when(i == 0)
    def _():
        num_ref[...] = jnp.zeros((G, D), jnp.float32)
        den_ref[...] = jnp.zeros((G, D), jnp.float32)

    num_ref[...] += num_part
    den_ref[...] += jnp.broadcast_to(den_part, (G, D))

    @pl.when(i == NBLK - 1)
    def _():
        xgn = num_ref[...] / (den_ref[...] + 1e-16)
        xg = xg_ref[...]
        z = (jnp.dot(xgn, wta_ref[...], preferred_element_type=jnp.float32)
             + jnp.dot(xg, wtb_ref[...], preferred_element_type=jnp.float32)
             + bt_ref[...])
        xgo_ref[...] = _leaky(z) + xg


def _pool(xn, batch3, gmax, xg, wf, bf, wg, bg, wta, wtb, bt):
    return pl.pallas_call(
        _pool_body,
        grid=(NBLK,),
        in_specs=[
            pl.BlockSpec((BN, D), lambda i: (i, 0)),
            pl.BlockSpec((1, 1, BN), lambda i: (i, 0, 0)),
            pl.BlockSpec((8, G), lambda i: (0, 0)),
            pl.BlockSpec((G, D), lambda i: (0, 0)),
            pl.BlockSpec((D, D), lambda i: (0, 0)),
            pl.BlockSpec((1, D), lambda i: (0, 0)),
            pl.BlockSpec((1, D), lambda i: (0, 0)),
            pl.BlockSpec((1, 1), lambda i: (0, 0)),
            pl.BlockSpec((D, D), lambda i: (0, 0)),
            pl.BlockSpec((D, D), lambda i: (0, 0)),
            pl.BlockSpec((1, D), lambda i: (0, 0)),
        ],
        out_specs=pl.BlockSpec((G, D), lambda i: (0, 0)),
        out_shape=jax.ShapeDtypeStruct((G, D), jnp.float32),
        scratch_shapes=[
            pltpu.VMEM((G, D), jnp.float32),
            pltpu.VMEM((G, D), jnp.float32),
        ],
    )(xn, batch3, gmax, xg, wf, bf, wg, bg, wta, wtb, bt)


def kernel(x, x_global, edge_attr, edge_index, batch_ind, num_graphs,
           Wm, bm, Wa, ba, Wg, bg, Wf, bf, Wt, bt):
    src = edge_index[0]
    dst = edge_index[1]
    batch3 = batch_ind.reshape(NBLK, 1, BN)

    we = [Wm[i][D:] for i in range(STEPS)]
    wx = [Wm[i][:D] for i in range(STEPS)]
    bm2 = [bm[i].reshape(1, D) for i in range(STEPS)]
    wax = [Wa[i][:D] for i in range(STEPS)]
    wag = [Wa[i][D:2 * D] for i in range(STEPS)]
    waa = [Wa[i][2 * D:] for i in range(STEPS)]
    ba2 = [ba[i].reshape(1, D) for i in range(STEPS)]
    wg2 = [Wg[i].reshape(1, D) for i in range(STEPS)]
    bg2 = [bg[i].reshape(1, 1) for i in range(STEPS)]
    wf2 = [Wf[i] for i in range(STEPS)]
    bf2 = [bf[i].reshape(1, D) for i in range(STEPS)]
    wta = [Wt[i][:D] for i in range(STEPS)]
    wtb = [Wt[i][D:] for i in range(STEPS)]
    bt2 = [bt[i].reshape(1, D) for i in range(STEPS)]

    eaw0, eaw1 = _eaw(edge_attr, we[0], we[1])
    xw0 = _xw(x, wx[0], bm2[0])

    agg0 = _sc_agg(xw0, eaw0, src, dst).reshape(N, D)
    x1, gmax0, xw1 = _update(
        x, agg0, batch3, x_global, wax[0], wag[0], waa[0], ba2[0],
        wg2[0], bg2[0], wx[1], bm2[1], True)
    xg1 = _pool(x1, batch3, gmax0, x_global, wf2[0], bf2[0], wg2[0], bg2[0],
                wta[0], wtb[0], bt2[0])

    agg1 = _sc_agg(xw1, eaw1, src, dst).reshape(N, D)
    x2, gmax1 = _update(
        x1, agg1, batch3, xg1, wax[1], wag[1], waa[1], ba2[1],
        wg2[1], bg2[1], wx[1], bm2[1], False)
    xg2 = _pool(x2, batch3, gmax1, xg1, wf2[1], bf2[1], wg2[1], bg2[1],
                wta[1], wtb[1], bt2[1])
    return (x2, xg2)


# speedup vs baseline: 1.0461x; 1.0461x over previous
import functools

import jax
import jax.numpy as jnp
from jax import lax
from jax.experimental import pallas as pl
from jax.experimental.pallas import tpu as pltpu
from jax.experimental.pallas import tpu_sc as plsc

N = 100000
E = 1600000
D = 32
DE = 16
G = 64
STEPS = 2

NC = 2
NS = 16
NWK = NC * NS
L = 16
NPW = N // NWK
W = 2560
NWIN = E // W
C = 96

NBLK = 25
BN = N // NBLK
EBLK = 200
BE = E // EBLK

_NEG_INF = float("-inf")


def _sc_agg_body(xw_hbm, eaw_hbm, src_hbm, dst_hbm, agg_hbm,
                 dstbuf, srcbuf, keptid, keptsrc, keptnl,
                 idxc, srcc, nlc, eawb, xwb, aggb,
                 sem_d, sem_s, sem_e, sem_x, sem_e2, sem_x2):
    cid = lax.axis_index("c")
    sid = lax.axis_index("s")
    wid = sid * NC + cid
    lo = wid * NPW
    io = lax.iota(jnp.int32, L)
    minf = jnp.full((L,), _NEG_INF, jnp.float32)
    zid = jnp.zeros((L,), jnp.int32)

    def _init_agg(r, _):
        aggb[r, pl.ds(0, L)] = minf
        aggb[r, pl.ds(L, L)] = minf
        return 0
    lax.fori_loop(0, NPW + 1, _init_agg, 0)

    def _init_ids(i, _):
        keptid[pl.ds(i * L, L)] = zid
        keptsrc[pl.ds(i * L, L)] = zid
        return 0
    lax.fori_loop(0, (W + L) // L, _init_ids, 0)

    def _issue(win):
        par = (win % 2) * W
        pltpu.async_copy(dst_hbm.at[pl.ds(win * W, W)],
                         dstbuf.at[pl.ds(par, W)], sem_d)
        pltpu.async_copy(src_hbm.at[pl.ds(win * W, W)],
                         srcbuf.at[pl.ds(par, W)], sem_s)

    def _wait(win):
        par = (win % 2) * W
        pltpu.make_async_copy(dst_hbm.at[pl.ds(win * W, W)],
                              dstbuf.at[pl.ds(par, W)], sem_d).wait()
        pltpu.make_async_copy(src_hbm.at[pl.ds(win * W, W)],
                              srcbuf.at[pl.ds(par, W)], sem_s).wait()

    def _gissue(slot, se, sx):
        pltpu.async_copy(eaw_hbm.at[idxc.at[slot]], eawb.at[slot], se)
        pltpu.async_copy(xw_hbm.at[srcc.at[slot]], xwb.at[slot], sx)

    def _gwait(slot, se, sx):
        pltpu.make_async_copy(eaw_hbm.at[idxc.at[slot]],
                              eawb.at[slot], se).wait()
        pltpu.make_async_copy(xw_hbm.at[srcc.at[slot]],
                              xwb.at[slot], sx).wait()

    def _consume(slot, nvalid):
        def _msg(r, _):
            for h in (0, L):
                ev = eawb[slot, r, pl.ds(h, L)]
                xv = xwb[slot, r, pl.ds(h, L)]
                sm = xv + ev
                eawb[slot, r, pl.ds(h, L)] = jnp.maximum(sm, 0.01 * sm)
            return 0
        plsc.parallel_loop(0, C, 1, unroll=4, carry=jnp.int32(0))(_msg)

        def _rmw_grp(g, _):
            nlr = nlc[slot, pl.ds(g * L, L)]
            valid = (g * L + io) < nvalid
            nlv = jnp.where(valid, nlr, NPW)
            for lane in range(L):
                nl = nlv[lane]
                row = g * L + lane
                for h in (0, L):
                    msg = eawb[slot, row, pl.ds(h, L)]
                    av = aggb[nl, pl.ds(h, L)]
                    aggb[nl, pl.ds(h, L)] = jnp.maximum(av, msg)
            return 0
        lax.fori_loop(0, C // L, _rmw_grp, 0)

    def _stage(slot, coff):
        for q in range(C // L):
            idxc[slot, pl.ds(q * L, L)] = keptid[pl.ds(coff + q * L, L)]
            srcc[slot, pl.ds(q * L, L)] = keptsrc[pl.ds(coff + q * L, L)]
            nlc[slot, pl.ds(q * L, L)] = keptnl[pl.ds(coff + q * L, L)]

    _issue(0)

    def _win_body(win, pcnt):
        wbase = win * W
        par = (win % 2) * W
        slot = win % 2
        pslot = 1 - slot
        _wait(win)

        @pl.when(win + 1 < NWIN)
        def _():
            _issue(win + 1)

        def _scan(v, cnt):
            dv = dstbuf[pl.ds(par + v * L, L)]
            sv = srcbuf[pl.ds(par + v * L, L)]
            m = (dv >= lo) & (dv < lo + NPW)
            plsc.store_compressed(keptid.at[pl.ds(cnt, L)],
                                  wbase + v * L + io, mask=m)
            plsc.store_compressed(keptsrc.at[pl.ds(cnt, L)], sv, mask=m)
            plsc.store_compressed(keptnl.at[pl.ds(cnt, L)], dv - lo, mask=m)
            pc = plsc.all_reduce_population_count(m)
            return cnt + pc[0]

        cnt = plsc.parallel_loop(0, W // L, 1, unroll=4,
                                 carry=jnp.int32(0))(_scan)

        _stage(slot, 0)

        @pl.when(pcnt > 0)
        def _():
            _gwait(pslot, sem_e, sem_x)
            _consume(pslot, jnp.minimum(pcnt, C))

        @pl.when(cnt > 0)
        def _():
            _gissue(slot, sem_e, sem_x)

        @pl.when(cnt > C)
        def _():
            nch = lax.div(cnt + (C - 1), C)

            def _chunk(c, _):
                coff = c * C
                _stage(pslot, coff)
                _gissue(pslot, sem_e2, sem_x2)
                _gwait(pslot, sem_e2, sem_x2)
                _consume(pslot, cnt - coff)
                return 0

            lax.fori_loop(1, nch, _chunk, 0)

        return cnt

    pcnt = lax.fori_loop(0, NWIN, _win_body, jnp.int32(0))

    @pl.when(pcnt > 0)
    def _():
        _gwait((NWIN - 1) % 2, sem_e, sem_x)
        _consume((NWIN - 1) % 2, jnp.minimum(pcnt, C))

    pltpu.sync_copy(aggb.at[pl.ds(0, NPW)], agg_hbm.at[wid])


def _sc_agg(xw, eaw, src, dst):
    mesh = plsc.VectorSubcoreMesh(
        core_axis_name="c", subcore_axis_name="s", num_cores=NC, num_subcores=NS)
    f = pl.kernel(
        _sc_agg_body,
        out_type=jax.ShapeDtypeStruct((NWK, NPW, D), jnp.float32),
        mesh=mesh,
        compiler_params=pltpu.CompilerParams(
            needs_layout_passes=False, use_tc_tiling_on_sc=False),
        scratch_types=[
            pltpu.VMEM((2 * W,), jnp.int32),
            pltpu.VMEM((2 * W,), jnp.int32),
            pltpu.VMEM((W + L,), jnp.int32),
            pltpu.VMEM((W + L,), jnp.int32),
            pltpu.VMEM((W + L,), jnp.int32),
            pltpu.VMEM((2, C), jnp.int32),
            pltpu.VMEM((2, C), jnp.int32),
            pltpu.VMEM((2, C), jnp.int32),
            pltpu.VMEM((2, C, D), jnp.float32),
            pltpu.VMEM((2, C, D), jnp.float32),
            pltpu.VMEM((NPW + 1, D), jnp.float32),
            pltpu.SemaphoreType.DMA,
            pltpu.SemaphoreType.DMA,
            pltpu.SemaphoreType.DMA,
            pltpu.SemaphoreType.DMA,
            pltpu.SemaphoreType.DMA,
            pltpu.SemaphoreType.DMA,
        ],
    )
    return f(xw, eaw, src, dst)


def _leaky(v):
    return jnp.maximum(v, 0.01 * v)


def _eaw_body(ea_ref, w0_ref, w1_ref, o0_ref, o1_ref):
    ea = ea_ref[...]
    o0_ref[...] = jnp.dot(ea, w0_ref[...], preferred_element_type=jnp.float32)
    o1_ref[...] = jnp.dot(ea, w1_ref[...], preferred_element_type=jnp.float32)


def _eaw(edge_attr, we0, we1):
    return pl.pallas_call(
        _eaw_body,
        grid=(EBLK,),
        in_specs=[
            pl.BlockSpec((BE, DE), lambda i: (i, 0)),
            pl.BlockSpec((DE, D), lambda i: (0, 0)),
            pl.BlockSpec((DE, D), lambda i: (0, 0)),
        ],
        out_specs=[
            pl.BlockSpec((BE, D), lambda i: (i, 0)),
            pl.BlockSpec((BE, D), lambda i: (i, 0)),
        ],
        out_shape=[
            jax.ShapeDtypeStruct((E, D), jnp.float32),
            jax.ShapeDtypeStruct((E, D), jnp.float32),
        ],
    )(edge_attr, we0, we1)


def _xw_body(x_ref, w_ref, b_ref, o_ref):
    o_ref[...] = (
        jnp.dot(x_ref[...], w_ref[...], preferred_element_type=jnp.float32)
        + b_ref[...])


def _xw(x, wx, b):
    return pl.pallas_call(
        _xw_body,
        grid=(NBLK,),
        in_specs=[
            pl.BlockSpec((BN, D), lambda i: (i, 0)),
            pl.BlockSpec((D, D), lambda i: (0, 0)),
            pl.BlockSpec((1, D), lambda i: (0, 0)),
        ],
        out_specs=pl.BlockSpec((BN, D), lambda i: (i, 0)),
        out_shape=jax.ShapeDtypeStruct((N, D), jnp.float32),
    )(x, wx, b)


def _update_body(x_ref, agg_ref, b_ref, xg_ref, wax_ref, wag_ref, waa_ref,
                 ba_ref, wg_ref, bg_ref, wxn_ref, bxn_ref,
                 xn_ref, gmax_ref, *maybe_xwn, with_next):
    xwn_ref = maybe_xwn[0] if with_next else None
    i = pl.program_id(0)
    x = x_ref[...]
    agg = agg_ref[...]
    agg = jnp.where(jnp.isfinite(agg), agg, 0.0)
    b = b_ref[0, 0]
    oh = (b[:, None] == lax.broadcasted_iota(jnp.int32, (BN, G), 1)
          ).astype(jnp.float32)
    xgb = jnp.dot(oh, xg_ref[...], preferred_element_type=jnp.float32)
    z2 = (jnp.dot(x, wax_ref[...], preferred_element_type=jnp.float32)
          + jnp.dot(xgb, wag_ref[...], preferred_element_type=jnp.float32)
          + jnp.dot(agg, waa_ref[...], preferred_element_type=jnp.float32)
          + ba_ref[...])
    xn = _leaky(z2) + x
    xn_ref[...] = xn
    gate = jnp.sum(xn * wg_ref[...], axis=1) + bg_ref[0, 0]
    gm_blk = jnp.max(jnp.where(oh > 0, gate[:, None], _NEG_INF), axis=0)

    @pl.when(i == 0)
    def _():
        gmax_ref[...] = jnp.full((8, G), _NEG_INF, jnp.float32)

    gmax_ref[...] = jnp.maximum(gmax_ref[...], jnp.broadcast_to(gm_blk, (8, G)))
    if with_next:
        xwn_ref[...] = (
            jnp.dot(xn, wxn_ref[...], preferred_element_type=jnp.float32)
            + bxn_ref[...])


def _update(x, agg, batch3, xg, wax, wag, waa, ba, wg, bg, wxn, bxn,
            with_next):
    outs = [
        jax.ShapeDtypeStruct((N, D), jnp.float32),
        jax.ShapeDtypeStruct((8, G), jnp.float32),
    ]
    out_specs = [
        pl.BlockSpec((BN, D), lambda i: (i, 0)),
        pl.BlockSpec((8, G), lambda i: (0, 0)),
    ]
    if with_next:
        outs.append(jax.ShapeDtypeStruct((N, D), jnp.float32))
        out_specs.append(pl.BlockSpec((BN, D), lambda i: (i, 0)))
    res = pl.pallas_call(
        functools.partial(_update_body, with_next=with_next),
        grid=(NBLK,),
        in_specs=[
            pl.BlockSpec((BN, D), lambda i: (i, 0)),
            pl.BlockSpec((BN, D), lambda i: (i, 0)),
            pl.BlockSpec((1, 1, BN), lambda i: (i, 0, 0)),
            pl.BlockSpec((G, D), lambda i: (0, 0)),
            pl.BlockSpec((D, D), lambda i: (0, 0)),
            pl.BlockSpec((D, D), lambda i: (0, 0)),
            pl.BlockSpec((D, D), lambda i: (0, 0)),
            pl.BlockSpec((1, D), lambda i: (0, 0)),
            pl.BlockSpec((1, D), lambda i: (0, 0)),
            pl.BlockSpec((1, 1), lambda i: (0, 0)),
            pl.BlockSpec((D, D), lambda i: (0, 0)),
            pl.BlockSpec((1, D), lambda i: (0, 0)),
        ],
        out_specs=out_specs,
        out_shape=outs,
    )(x, agg, batch3, xg, wax, wag, waa, ba, wg, bg, wxn, bxn)
    return res


def _pool_body(xn_ref, b_ref, gmax_ref, xg_ref, wf_ref, bf_ref, wg_ref,
               bg_ref, wta_ref, wtb_ref, bt_ref, xgo_ref, num_ref, den_ref):
    i = pl.program_id(0)
    xn = xn_ref[...]
    b = b_ref[0, 0]
    oh = (b[:, None] == lax.broadcasted_iota(jnp.int32, (BN, G), 1)
          ).astype(jnp.float32)
    gate = jnp.sum(xn * wg_ref[...], axis=1) + bg_ref[0, 0]
    gm = gmax_ref[...][0]
    gm = jnp.where(gm == _NEG_INF, 0.0, gm)
    gnode = jnp.dot(oh, gm[:, None], preferred_element_type=jnp.float32)[:, 0]
    e = jnp.exp(gate - gnode)
    feat = _leaky(
        jnp.dot(xn, wf_ref[...], preferred_element_type=jnp.float32)
        + bf_ref[...])
    ef = e[:, None] * feat
    num_part = lax.dot_general(oh, ef, (((0,), (0,)), ((), ())),
                               preferred_element_type=jnp.float32)
    den_part = lax.dot_general(oh, e[:, None], (((0,), (0,)), ((), ())),
                               preferred_element_type=jnp.float32)

    @pl.when(i == 0)
    def _():
        num_ref[...] = jnp.zeros((G, D), jnp.float32)
        den_ref[...] = jnp.zeros((G, D), jnp.float32)

    num_ref[...] += num_part
    den_ref[...] += jnp.broadcast_to(den_part, (G, D))

    @pl.when(i == NBLK - 1)
    def _():
        xgn = num_ref[...] / (den_ref[...] + 1e-16)
        xg = xg_ref[...]
        z = (jnp.dot(xgn, wta_ref[...], preferred_element_type=jnp.float32)
             + jnp.dot(xg, wtb_ref[...], preferred_element_type=jnp.float32)
             + bt_ref[...])
        xgo_ref[...] = _leaky(z) + xg


def _pool(xn, batch3, gmax, xg, wf, bf, wg, bg, wta, wtb, bt):
    return pl.pallas_call(
        _pool_body,
        grid=(NBLK,),
        in_specs=[
            pl.BlockSpec((BN, D), lambda i: (i, 0)),
            pl.BlockSpec((1, 1, BN), lambda i: (i, 0, 0)),
            pl.BlockSpec((8, G), lambda i: (0, 0)),
            pl.BlockSpec((G, D), lambda i: (0, 0)),
            pl.BlockSpec((D, D), lambda i: (0, 0)),
            pl.BlockSpec((1, D), lambda i: (0, 0)),
            pl.BlockSpec((1, D), lambda i: (0, 0)),
            pl.BlockSpec((1, 1), lambda i: (0, 0)),
            pl.BlockSpec((D, D), lambda i: (0, 0)),
            pl.BlockSpec((D, D), lambda i: (0, 0)),
            pl.BlockSpec((1, D), lambda i: (0, 0)),
        ],
        out_specs=pl.BlockSpec((G, D), lambda i: (0, 0)),
        out_shape=jax.ShapeDtypeStruct((G, D), jnp.float32),
        scratch_shapes=[
            pltpu.VMEM((G, D), jnp.float32),
            pltpu.VMEM((G, D), jnp.float32),
        ],
    )(xn, batch3, gmax, xg, wf, bf, wg, bg, wta, wtb, bt)


def kernel(x, x_global, edge_attr, edge_index, batch_ind, num_graphs,
           Wm, bm, Wa, ba, Wg, bg, Wf, bf, Wt, bt):
    src = edge_index[0]
    dst = edge_index[1]
    batch3 = batch_ind.reshape(NBLK, 1, BN)

    we = [Wm[i][D:] for i in range(STEPS)]
    wx = [Wm[i][:D] for i in range(STEPS)]
    bm2 = [bm[i].reshape(1, D) for i in range(STEPS)]
    wax = [Wa[i][:D] for i in range(STEPS)]
    wag = [Wa[i][D:2 * D] for i in range(STEPS)]
    waa = [Wa[i][2 * D:] for i in range(STEPS)]
    ba2 = [ba[i].reshape(1, D) for i in range(STEPS)]
    wg2 = [Wg[i].reshape(1, D) for i in range(STEPS)]
    bg2 = [bg[i].reshape(1, 1) for i in range(STEPS)]
    wf2 = [Wf[i] for i in range(STEPS)]
    bf2 = [bf[i].reshape(1, D) for i in range(STEPS)]
    wta = [Wt[i][:D] for i in range(STEPS)]
    wtb = [Wt[i][D:] for i in range(STEPS)]
    bt2 = [bt[i].reshape(1, D) for i in range(STEPS)]

    eaw0, eaw1 = _eaw(edge_attr, we[0], we[1])
    xw0 = _xw(x, wx[0], bm2[0])

    agg0 = _sc_agg(xw0, eaw0, src, dst).reshape(N, D)
    x1, gmax0, xw1 = _update(
        x, agg0, batch3, x_global, wax[0], wag[0], waa[0], ba2[0],
        wg2[0], bg2[0], wx[1], bm2[1], True)
    xg1 = _pool(x1, batch3, gmax0, x_global, wf2[0], bf2[0], wg2[0], bg2[0],
                wta[0], wtb[0], bt2[0])

    agg1 = _sc_agg(xw1, eaw1, src, dst).reshape(N, D)
    x2, gmax1 = _update(
        x1, agg1, batch3, xg1, wax[1], wag[1], waa[1], ba2[1],
        wg2[1], bg2[1], wx[1], bm2[1], False)
    xg2 = _pool(x2, batch3, gmax1, xg1, wf2[1], bf2[1], wg2[1], bg2[1],
                wta[1], wtb[1], bt2[1])
    return (x2, xg2)
